# baseline (speedup 1.0000x reference)
import functools

import jax
import jax.numpy as jnp
from jax import lax
from jax.experimental import pallas as pl
from jax.experimental.pallas import tpu as pltpu
from jax.experimental.pallas import tpu_sc as plsc

NC = 2
NS = 16
L = 16
NW = NC * NS

CB = 32
IDX_W = 128
P = 80


def _sigmoid(t):
    return 1.0 / (1.0 + jnp.exp(-t))


@functools.lru_cache(maxsize=None)
def _build_transpose(V, D):
    FULL = V // 128
    REM = V - FULL * 128
    NK = -(-FULL // NW)
    if NK % 2:
        NK += 1
    OB = 128 * P
    mesh = plsc.VectorSubcoreMesh(core_axis_name="c", subcore_axis_name="s")

    @functools.partial(
        pl.kernel,
        out_type=(
            jax.ShapeDtypeStruct((V * P,), jnp.float32),
            jax.ShapeDtypeStruct((V * P,), jnp.float32),
        ),
        mesh=mesh,
        compiler_params=pltpu.CompilerParams(
            needs_layout_passes=False, use_tc_tiling_on_sc=True),
        scratch_types=[
            pltpu.VMEM((D, 128), jnp.float32),
            pltpu.VMEM((D, 128), jnp.float32),
            pltpu.VMEM((128 * P,), jnp.float32),
            pltpu.VMEM((128 * P,), jnp.float32),
            pltpu.SemaphoreType.DMA,
            pltpu.SemaphoreType.DMA,
            pltpu.SemaphoreType.DMA,
            pltpu.SemaphoreType.DMA,
        ],
    )
    def tr_kernel(ctT, xtT, tail_c, tail_x, out_c, out_x,
                  blk0, blk1, ob0, ob1, semr0, semr1, semw0, semw1):
        blk = (blk0, blk1)
        obuf = (ob0, ob1)
        semr = (semr0, semr1)
        semw = (semw0, semw1)
        wid = lax.axis_index("s") * NC + lax.axis_index("c")
        lane = lax.iota(jnp.int32, L)
        OB = 128 * P

        @pl.when(wid == 0)
        def _():
            pltpu.sync_copy(tail_c, out_c.at[pl.ds(FULL * OB, REM * P)])
            pltpu.sync_copy(tail_x, out_x.at[pl.ds(FULL * OB, REM * P)])

        jvecs = [(9 * lane + 16 * t) & 127 for t in range(8)]
        jvP = [jv * P for jv in jvecs]

        def col_of(kv):
            return wid + kv * NW

        def process_table(tbl, out):
            def fire_read(kv, b):
                c = col_of(kv)

                @pl.when(c < FULL)
                def _():
                    for r in range(D // 8):
                        pltpu.async_copy(
                            tbl.at[pl.ds(r * 8, 8),
                                   pl.ds(pl.multiple_of(c * 128, 128), 128)],
                            blk[b].at[pl.ds(r * 8, 8), :], semr[b])

            def pair_body(k, _):
                for b in range(2):
                    kv = k * 2 + b
                    c = col_of(kv)

                    @pl.when(c < FULL)
                    def _():
                        for r in range(D // 8):
                            pltpu.make_async_copy(
                                tbl.at[pl.ds(0, 8), pl.ds(0, 128)],
                                blk[b].at[pl.ds(r * 8, 8), :],
                                semr[b]).wait()

                        @pl.when(k >= 1)
                        def _():
                            pltpu.make_async_copy(
                                obuf[b], out.at[pl.ds(0, OB)],
                                semw[b]).wait()

                        DSU = 8

                        def d_scatter(dd, _):
                            for u in range(DSU):
                                d = dd * DSU + u
                                dsp = jnp.full((L,), d, jnp.int32)
                                for t in range(8):
                                    v = plsc.load_gather(
                                        blk[b], [dsp, jvecs[t]])
                                    plsc.store_scatter(
                                        obuf[b], [jvP[t] + d], v)
                            return ()

                        lax.fori_loop(0, D // DSU, d_scatter, ())
                        pltpu.async_copy(
                            obuf[b],
                            out.at[pl.ds(pl.multiple_of(c * OB, 128), OB)],
                            semw[b])

                    fire_read(kv + 2, b)
                return ()

            fire_read(0, 0)
            fire_read(1, 1)
            lax.fori_loop(0, NK // 2, pair_body, ())
            for b in range(2):
                pltpu.make_async_copy(
                    obuf[b], out.at[pl.ds(0, OB)], semw[b]).wait()

        process_table(ctT, out_c)
        process_table(xtT, out_x)

    return tr_kernel


@functools.lru_cache(maxsize=None)
def _build(V, D, B, NNEG):
    assert B % (NW * CB) == 0 and D % L == 0 and D & (D - 1) == 0
    bw = B // NW
    nchunk = bw // CB
    nneg_rows = CB * NNEG
    nj = nneg_rows // IDX_W
    assert nneg_rows % IDX_W == 0
    ngroups = CB // L

    mesh = plsc.VectorSubcoreMesh(core_axis_name="c", subcore_axis_name="s")

    @functools.partial(
        pl.kernel,
        out_type=(
            jax.ShapeDtypeStruct((B,), jnp.float32),
            jax.ShapeDtypeStruct((B * NNEG,), jnp.float32),
        ),
        mesh=mesh,
        compiler_params=pltpu.CompilerParams(
            needs_layout_passes=False, use_tc_tiling_on_sc=False),
        scratch_types=[
            pltpu.VMEM((2, CB), jnp.int32),
            pltpu.VMEM((2, CB), jnp.int32),
            pltpu.VMEM((2, nneg_rows), jnp.int32),
            pltpu.VMEM((2, CB, P), jnp.float32),
            pltpu.VMEM((2, CB, P), jnp.float32),
            pltpu.VMEM((2, nneg_rows, P), jnp.float32),
            pltpu.VMEM((2, CB), jnp.float32),
            pltpu.VMEM((2, nneg_rows), jnp.float32),
            pltpu.SemaphoreType.DMA,
            pltpu.SemaphoreType.DMA,
            pltpu.SemaphoreType.DMA,
            pltpu.SemaphoreType.DMA,
            pltpu.SemaphoreType.DMA,
            pltpu.SemaphoreType.DMA,
        ],
    )
    def sc_kernel(ct_hbm, xt_hbm, cw_hbm, xw_hbm, nw_hbm,
                  pos_hbm, neg_hbm,
                  idxc, idxx, idxn, crows, xrows, nrows, posb, negb,
                  semi0, semi1, semr0, semr1, semo0, semo1):
        semi = (semi0, semi1)
        semr = (semr0, semr1)
        semo = (semo0, semo1)
        wid = lax.axis_index("s") * NC + lax.axis_index("c")
        lane = lax.iota(jnp.int32, L)

        def chunk_base(i):
            return pl.multiple_of(wid * bw + i * CB, CB)

        def fire_idx(i):
            s = i % 2
            base = chunk_base(i)
            nbase = pl.multiple_of(base * NNEG, CB * NNEG)
            return [
                pltpu.async_copy(cw_hbm.at[pl.ds(base, CB)],
                                 idxc.at[s], semi[s]),
                pltpu.async_copy(xw_hbm.at[pl.ds(base, CB)],
                                 idxx.at[s], semi[s]),
                pltpu.async_copy(nw_hbm.at[pl.ds(nbase, nneg_rows)],
                                 idxn.at[s], semi[s]),
            ]

        def fire_rows(i):
            s = i % 2
            cps = [
                pltpu.async_copy(ct_hbm.at[idxc.at[s]], crows.at[s], semr[s]),
                pltpu.async_copy(xt_hbm.at[idxx.at[s]], xrows.at[s], semr[s]),
            ]
            for j in range(nj):
                cps.append(pltpu.async_copy(
                    xt_hbm.at[idxn.at[s, pl.ds(j * IDX_W, IDX_W)]],
                    nrows.at[s, pl.ds(j * IDX_W, IDX_W)], semr[s]))
            return cps

        def compute(i):
            s = i % 2
            cr, xr, nr = crows.at[s], xrows.at[s], nrows.at[s]

            def g_body(g, _):
                cidx = lane + g * L
                nrow0 = (lane + g * L) * NNEG

                def d_body(d, carry):
                    accp = carry[0]
                    accs = carry[1]
                    dsp = (jnp.full((L,), d, jnp.int32) + lane) & (D - 1)
                    c = plsc.load_gather(cr, [cidx, dsp])
                    x = plsc.load_gather(xr, [cidx, dsp])
                    accp = accp + c * x
                    accs = tuple(
                        accs[n]
                        + plsc.load_gather(nr, [nrow0 + n, dsp]) * c
                        for n in range(NNEG))
                    return (accp, accs)

                zero = jnp.zeros((L,), jnp.float32)
                accp, accs = lax.fori_loop(
                    0, D, d_body, (zero, (zero,) * NNEG))
                plsc.store_scatter(posb.at[s], [cidx], _sigmoid(accp))
                for n in range(NNEG):
                    plsc.store_scatter(negb.at[s], [nrow0 + n],
                                       _sigmoid(-accs[n]))
                return ()

            lax.fori_loop(0, ngroups, g_body, ())

        def fire_out(i):
            s = i % 2
            base = chunk_base(i)
            nbase = pl.multiple_of(base * NNEG, CB * NNEG)
            return [
                pltpu.async_copy(posb.at[s], pos_hbm.at[pl.ds(base, CB)],
                                 semo[s]),
                pltpu.async_copy(negb.at[s],
                                 neg_hbm.at[pl.ds(nbase, nneg_rows)],
                                 semo[s]),
            ]

        def drain(cps):
            for cp in cps:
                cp.wait()

        d_idx, d_rows, d_out = {}, {}, {}
        d_idx[0] = fire_idx(0)
        d_idx[1] = fire_idx(1)
        drain(d_idx.pop(0))
        d_rows[0] = fire_rows(0)
        for i in range(nchunk):
            if i + 1 < nchunk:
                drain(d_idx.pop(i + 1))
                d_rows[i + 1] = fire_rows(i + 1)
            drain(d_rows.pop(i))
            if i + 2 < nchunk:
                d_idx[i + 2] = fire_idx(i + 2)
            if i - 2 in d_out:
                drain(d_out.pop(i - 2))
            compute(i)
            d_out[i] = fire_out(i)
        for k in sorted(d_out):
            drain(d_out[k])

    return sc_kernel


def kernel(center_table, context_table, center_words, context_words,
           negative_words):
    V, D = center_table.shape
    B = center_words.shape[0]
    NNEG = negative_words.shape[1]
    FULL = V // 128
    pad = ((0, 0), (0, P - D))
    tail_c = jnp.pad(center_table[FULL * 128:], pad).reshape(-1)
    tail_x = jnp.pad(context_table[FULL * 128:], pad).reshape(-1)
    tr = _build_transpose(V, D)
    lin_c, lin_x = tr(center_table.T, context_table.T, tail_c, tail_x)
    fn = _build(V, D, B, NNEG)
    pos, neg_flat = fn(
        lin_c.reshape(V, P),
        lin_x.reshape(V, P),
        center_words.astype(jnp.int32),
        context_words.astype(jnp.int32),
        negative_words.astype(jnp.int32).reshape(-1),
    )
    return pos, neg_flat.reshape(B, NNEG)


# speedup vs baseline: 1.0079x; 1.0079x over previous
import functools

import jax
import jax.numpy as jnp
from jax import lax
from jax.experimental import pallas as pl
from jax.experimental.pallas import tpu as pltpu
from jax.experimental.pallas import tpu_sc as plsc

NC = 2
NS = 16
L = 16
NW = NC * NS

CB = 32
IDX_W = 128
P = 80


def _sigmoid(t):
    return 1.0 / (1.0 + jnp.exp(-t))


@functools.lru_cache(maxsize=None)
def _build_transpose(V, D):
    FULL = V // 128
    REM = V - FULL * 128
    NK = -(-FULL // NW)
    if NK % 2:
        NK += 1
    OB = 128 * P
    mesh = plsc.VectorSubcoreMesh(core_axis_name="c", subcore_axis_name="s")

    @functools.partial(
        pl.kernel,
        out_type=(
            jax.ShapeDtypeStruct((V * P,), jnp.float32),
            jax.ShapeDtypeStruct((V * P,), jnp.float32),
        ),
        mesh=mesh,
        compiler_params=pltpu.CompilerParams(
            needs_layout_passes=False, use_tc_tiling_on_sc=True),
        scratch_types=[
            pltpu.VMEM((D, 128), jnp.float32),
            pltpu.VMEM((D, 128), jnp.float32),
            pltpu.VMEM((128 * P,), jnp.float32),
            pltpu.VMEM((128 * P,), jnp.float32),
            pltpu.SemaphoreType.DMA,
            pltpu.SemaphoreType.DMA,
            pltpu.SemaphoreType.DMA,
            pltpu.SemaphoreType.DMA,
        ],
    )
    def tr_kernel(ctT, xtT, tail_c, tail_x, out_c, out_x,
                  blk0, blk1, ob0, ob1, semr0, semr1, semw0, semw1):
        blk = (blk0, blk1)
        obuf = (ob0, ob1)
        semr = (semr0, semr1)
        semw = (semw0, semw1)
        wid = lax.axis_index("s") * NC + lax.axis_index("c")
        lane = lax.iota(jnp.int32, L)
        OB = 128 * P

        @pl.when(wid == 0)
        def _():
            pltpu.sync_copy(tail_c, out_c.at[pl.ds(FULL * OB, REM * P)])
            pltpu.sync_copy(tail_x, out_x.at[pl.ds(FULL * OB, REM * P)])

        jvecs = [(9 * lane + 16 * t) & 127 for t in range(8)]
        jvP = [jv * P for jv in jvecs]

        def col_of(kv):
            return wid + kv * NW

        def process_table(tbl, out):
            def fire_read(kv, b):
                c = col_of(kv)

                @pl.when(c < FULL)
                def _():
                    pltpu.async_copy(
                        tbl.at[:, pl.ds(pl.multiple_of(c * 128, 128), 128)],
                        blk[b], semr[b])

            def pair_body(k, _):
                for b in range(2):
                    kv = k * 2 + b
                    c = col_of(kv)

                    @pl.when(c < FULL)
                    def _():
                        pltpu.make_async_copy(
                            tbl.at[:, pl.ds(0, 128)], blk[b],
                            semr[b]).wait()

                        @pl.when(k >= 1)
                        def _():
                            pltpu.make_async_copy(
                                obuf[b], out.at[pl.ds(0, OB)],
                                semw[b]).wait()

                        DSU = 8

                        def d_scatter(dd, _):
                            for u in range(DSU):
                                d = dd * DSU + u
                                dsp = jnp.full((L,), d, jnp.int32)
                                for t in range(8):
                                    v = plsc.load_gather(
                                        blk[b], [dsp, jvecs[t]])
                                    plsc.store_scatter(
                                        obuf[b], [jvP[t] + d], v)
                            return ()

                        lax.fori_loop(0, D // DSU, d_scatter, ())
                        pltpu.async_copy(
                            obuf[b],
                            out.at[pl.ds(pl.multiple_of(c * OB, 128), OB)],
                            semw[b])

                    fire_read(kv + 2, b)
                return ()

            fire_read(0, 0)
            fire_read(1, 1)
            lax.fori_loop(0, NK // 2, pair_body, ())
            for b in range(2):
                pltpu.make_async_copy(
                    obuf[b], out.at[pl.ds(0, OB)], semw[b]).wait()

        process_table(ctT, out_c)
        process_table(xtT, out_x)

    return tr_kernel


@functools.lru_cache(maxsize=None)
def _build(V, D, B, NNEG):
    assert B % (NW * CB) == 0 and D % L == 0 and D & (D - 1) == 0
    bw = B // NW
    nchunk = bw // CB
    nneg_rows = CB * NNEG
    nj = nneg_rows // IDX_W
    assert nneg_rows % IDX_W == 0
    ngroups = CB // L

    mesh = plsc.VectorSubcoreMesh(core_axis_name="c", subcore_axis_name="s")

    @functools.partial(
        pl.kernel,
        out_type=(
            jax.ShapeDtypeStruct((B,), jnp.float32),
            jax.ShapeDtypeStruct((B * NNEG,), jnp.float32),
        ),
        mesh=mesh,
        compiler_params=pltpu.CompilerParams(
            needs_layout_passes=False, use_tc_tiling_on_sc=False),
        scratch_types=[
            pltpu.VMEM((2, CB), jnp.int32),
            pltpu.VMEM((2, CB), jnp.int32),
            pltpu.VMEM((2, nneg_rows), jnp.int32),
            pltpu.VMEM((2, CB, P), jnp.float32),
            pltpu.VMEM((2, CB, P), jnp.float32),
            pltpu.VMEM((2, nneg_rows, P), jnp.float32),
            pltpu.VMEM((2, CB), jnp.float32),
            pltpu.VMEM((2, nneg_rows), jnp.float32),
            pltpu.SemaphoreType.DMA,
            pltpu.SemaphoreType.DMA,
            pltpu.SemaphoreType.DMA,
            pltpu.SemaphoreType.DMA,
            pltpu.SemaphoreType.DMA,
            pltpu.SemaphoreType.DMA,
        ],
    )
    def sc_kernel(ct_hbm, xt_hbm, cw_hbm, xw_hbm, nw_hbm,
                  pos_hbm, neg_hbm,
                  idxc, idxx, idxn, crows, xrows, nrows, posb, negb,
                  semi0, semi1, semr0, semr1, semo0, semo1):
        semi = (semi0, semi1)
        semr = (semr0, semr1)
        semo = (semo0, semo1)
        wid = lax.axis_index("s") * NC + lax.axis_index("c")
        lane = lax.iota(jnp.int32, L)

        def chunk_base(i):
            return pl.multiple_of(wid * bw + i * CB, CB)

        def fire_idx(i):
            s = i % 2
            base = chunk_base(i)
            nbase = pl.multiple_of(base * NNEG, CB * NNEG)
            return [
                pltpu.async_copy(cw_hbm.at[pl.ds(base, CB)],
                                 idxc.at[s], semi[s]),
                pltpu.async_copy(xw_hbm.at[pl.ds(base, CB)],
                                 idxx.at[s], semi[s]),
                pltpu.async_copy(nw_hbm.at[pl.ds(nbase, nneg_rows)],
                                 idxn.at[s], semi[s]),
            ]

        def fire_rows(i):
            s = i % 2
            cps = [
                pltpu.async_copy(ct_hbm.at[idxc.at[s]], crows.at[s], semr[s]),
                pltpu.async_copy(xt_hbm.at[idxx.at[s]], xrows.at[s], semr[s]),
            ]
            for j in range(nj):
                cps.append(pltpu.async_copy(
                    xt_hbm.at[idxn.at[s, pl.ds(j * IDX_W, IDX_W)]],
                    nrows.at[s, pl.ds(j * IDX_W, IDX_W)], semr[s]))
            return cps

        def compute(i):
            s = i % 2
            cr, xr, nr = crows.at[s], xrows.at[s], nrows.at[s]

            def g_body(g, _):
                cidx = lane + g * L
                nrow0 = (lane + g * L) * NNEG

                def d_body(d, carry):
                    accp = carry[0]
                    accs = carry[1]
                    dsp = (jnp.full((L,), d, jnp.int32) + lane) & (D - 1)
                    c = plsc.load_gather(cr, [cidx, dsp])
                    x = plsc.load_gather(xr, [cidx, dsp])
                    accp = accp + c * x
                    accs = tuple(
                        accs[n]
                        + plsc.load_gather(nr, [nrow0 + n, dsp]) * c
                        for n in range(NNEG))
                    return (accp, accs)

                zero = jnp.zeros((L,), jnp.float32)
                accp, accs = lax.fori_loop(
                    0, D, d_body, (zero, (zero,) * NNEG))
                plsc.store_scatter(posb.at[s], [cidx], _sigmoid(accp))
                for n in range(NNEG):
                    plsc.store_scatter(negb.at[s], [nrow0 + n],
                                       _sigmoid(-accs[n]))
                return ()

            lax.fori_loop(0, ngroups, g_body, ())

        def fire_out(i):
            s = i % 2
            base = chunk_base(i)
            nbase = pl.multiple_of(base * NNEG, CB * NNEG)
            return [
                pltpu.async_copy(posb.at[s], pos_hbm.at[pl.ds(base, CB)],
                                 semo[s]),
                pltpu.async_copy(negb.at[s],
                                 neg_hbm.at[pl.ds(nbase, nneg_rows)],
                                 semo[s]),
            ]

        def drain(cps):
            for cp in cps:
                cp.wait()

        d_idx, d_rows, d_out = {}, {}, {}
        d_idx[0] = fire_idx(0)
        d_idx[1] = fire_idx(1)
        drain(d_idx.pop(0))
        d_rows[0] = fire_rows(0)
        for i in range(nchunk):
            if i + 1 < nchunk:
                drain(d_idx.pop(i + 1))
                d_rows[i + 1] = fire_rows(i + 1)
            drain(d_rows.pop(i))
            if i + 2 < nchunk:
                d_idx[i + 2] = fire_idx(i + 2)
            if i - 2 in d_out:
                drain(d_out.pop(i - 2))
            compute(i)
            d_out[i] = fire_out(i)
        for k in sorted(d_out):
            drain(d_out[k])

    return sc_kernel


def kernel(center_table, context_table, center_words, context_words,
           negative_words):
    V, D = center_table.shape
    B = center_words.shape[0]
    NNEG = negative_words.shape[1]
    FULL = V // 128
    pad = ((0, 0), (0, P - D))
    tail_c = jnp.pad(center_table[FULL * 128:], pad).reshape(-1)
    tail_x = jnp.pad(context_table[FULL * 128:], pad).reshape(-1)
    tr = _build_transpose(V, D)
    lin_c, lin_x = tr(center_table.T, context_table.T, tail_c, tail_x)
    fn = _build(V, D, B, NNEG)
    pos, neg_flat = fn(
        lin_c.reshape(V, P),
        lin_x.reshape(V, P),
        center_words.astype(jnp.int32),
        context_words.astype(jnp.int32),
        negative_words.astype(jnp.int32).reshape(-1),
    )
    return pos, neg_flat.reshape(B, NNEG)


# speedup vs baseline: 1.4295x; 1.4183x over previous
import functools

import jax
import jax.numpy as jnp
from jax import lax
from jax.experimental import pallas as pl
from jax.experimental.pallas import tpu as pltpu
from jax.experimental.pallas import tpu_sc as plsc

NC = 2
NS = 16
L = 16
NW = NC * NS

CB = 32
IDX_W = 128
P = 80


def _sigmoid(t):
    return 1.0 / (1.0 + jnp.exp(-t))


@functools.lru_cache(maxsize=None)
def _build_transpose(V, D):
    FULL = V // 128
    REM = V - FULL * 128
    NK = -(-FULL // NW)
    if NK % 2:
        NK += 1
    OB = 128 * P
    mesh = plsc.VectorSubcoreMesh(core_axis_name="c", subcore_axis_name="s")

    @functools.partial(
        pl.kernel,
        out_type=jax.ShapeDtypeStruct((V * P,), jnp.float32),
        mesh=mesh,
        compiler_params=pltpu.CompilerParams(
            needs_layout_passes=False, use_tc_tiling_on_sc=True),
        scratch_types=[
            pltpu.VMEM((D, 128), jnp.float32),
            pltpu.VMEM((D, 128), jnp.float32),
            pltpu.VMEM((128 * P,), jnp.float32),
            pltpu.VMEM((128 * P,), jnp.float32),
            pltpu.SemaphoreType.DMA,
            pltpu.SemaphoreType.DMA,
            pltpu.SemaphoreType.DMA,
            pltpu.SemaphoreType.DMA,
        ],
    )
    def tr_kernel(xtT, tail_x, out_x,
                  blk0, blk1, ob0, ob1, semr0, semr1, semw0, semw1):
        blk = (blk0, blk1)
        obuf = (ob0, ob1)
        semr = (semr0, semr1)
        semw = (semw0, semw1)
        wid = lax.axis_index("s") * NC + lax.axis_index("c")
        lane = lax.iota(jnp.int32, L)
        OB = 128 * P

        @pl.when(wid == 0)
        def _():
            pltpu.sync_copy(tail_x, out_x.at[pl.ds(FULL * OB, REM * P)])

        jvecs = [(9 * lane + 16 * t) & 127 for t in range(8)]
        jvP = [jv * P for jv in jvecs]

        def col_of(kv):
            return wid + kv * NW

        def process_table(tbl, out):
            def fire_read(kv, b):
                c = col_of(kv)

                @pl.when(c < FULL)
                def _():
                    pltpu.async_copy(
                        tbl.at[:, pl.ds(pl.multiple_of(c * 128, 128), 128)],
                        blk[b], semr[b])

            def pair_body(k, _):
                for b in range(2):
                    kv = k * 2 + b
                    c = col_of(kv)

                    @pl.when(c < FULL)
                    def _():
                        pltpu.make_async_copy(
                            tbl.at[:, pl.ds(0, 128)], blk[b],
                            semr[b]).wait()

                        @pl.when(k >= 1)
                        def _():
                            pltpu.make_async_copy(
                                obuf[b], out.at[pl.ds(0, OB)],
                                semw[b]).wait()

                        DSU = 8

                        def d_scatter(dd, _):
                            for u in range(DSU):
                                d = dd * DSU + u
                                dsp = jnp.full((L,), d, jnp.int32)
                                for t in range(8):
                                    v = plsc.load_gather(
                                        blk[b], [dsp, jvecs[t]])
                                    plsc.store_scatter(
                                        obuf[b], [jvP[t] + d], v)
                            return ()

                        lax.fori_loop(0, D // DSU, d_scatter, ())
                        pltpu.async_copy(
                            obuf[b],
                            out.at[pl.ds(pl.multiple_of(c * OB, 128), OB)],
                            semw[b])

                    fire_read(kv + 2, b)
                return ()

            fire_read(0, 0)
            fire_read(1, 1)
            lax.fori_loop(0, NK // 2, pair_body, ())
            for b in range(2):
                pltpu.make_async_copy(
                    obuf[b], out.at[pl.ds(0, OB)], semw[b]).wait()

        process_table(xtT, out_x)

    return tr_kernel


@functools.lru_cache(maxsize=None)
def _build(V, D, B, NNEG):
    assert B % (NW * CB) == 0 and D % L == 0 and D & (D - 1) == 0
    bw = B // NW
    nchunk = bw // CB
    nneg_rows = CB * NNEG
    nj = nneg_rows // IDX_W
    assert nneg_rows % IDX_W == 0
    ngroups = CB // L

    mesh = plsc.VectorSubcoreMesh(core_axis_name="c", subcore_axis_name="s")

    @functools.partial(
        pl.kernel,
        out_type=(
            jax.ShapeDtypeStruct((B,), jnp.float32),
            jax.ShapeDtypeStruct((B * NNEG,), jnp.float32),
        ),
        mesh=mesh,
        compiler_params=pltpu.CompilerParams(
            needs_layout_passes=False, use_tc_tiling_on_sc=False),
        scratch_types=[
            pltpu.VMEM((2, CB), jnp.int32),
            pltpu.VMEM((2, CB), jnp.int32),
            pltpu.VMEM((2, nneg_rows), jnp.int32),
            pltpu.VMEM((2, CB, D), jnp.float32),
            pltpu.VMEM((2, CB, P), jnp.float32),
            pltpu.VMEM((2, nneg_rows, P), jnp.float32),
            pltpu.VMEM((2, CB), jnp.float32),
            pltpu.VMEM((2, nneg_rows), jnp.float32),
            pltpu.SemaphoreType.DMA,
            pltpu.SemaphoreType.DMA,
            pltpu.SemaphoreType.DMA,
            pltpu.SemaphoreType.DMA,
            pltpu.SemaphoreType.DMA,
            pltpu.SemaphoreType.DMA,
        ],
    )
    def sc_kernel(ct_hbm, xt_hbm, cw_hbm, xw_hbm, nw_hbm,
                  pos_hbm, neg_hbm,
                  idxc, idxx, idxn, crows, xrows, nrows, posb, negb,
                  semi0, semi1, semr0, semr1, semo0, semo1):
        semi = (semi0, semi1)
        semr = (semr0, semr1)
        semo = (semo0, semo1)
        wid = lax.axis_index("s") * NC + lax.axis_index("c")
        lane = lax.iota(jnp.int32, L)

        def chunk_base(i):
            return pl.multiple_of(wid * bw + i * CB, CB)

        def fire_idx(i):
            s = i % 2
            base = chunk_base(i)
            nbase = pl.multiple_of(base * NNEG, CB * NNEG)
            return [
                pltpu.async_copy(cw_hbm.at[pl.ds(base, CB)],
                                 idxc.at[s], semi[s]),
                pltpu.async_copy(xw_hbm.at[pl.ds(base, CB)],
                                 idxx.at[s], semi[s]),
                pltpu.async_copy(nw_hbm.at[pl.ds(nbase, nneg_rows)],
                                 idxn.at[s], semi[s]),
            ]

        def fire_rows(i):
            s = i % 2
            cps = [
                pltpu.async_copy(ct_hbm.at[idxc.at[s]], crows.at[s], semr[s]),
                pltpu.async_copy(xt_hbm.at[idxx.at[s]], xrows.at[s], semr[s]),
            ]
            for j in range(nj):
                cps.append(pltpu.async_copy(
                    xt_hbm.at[idxn.at[s, pl.ds(j * IDX_W, IDX_W)]],
                    nrows.at[s, pl.ds(j * IDX_W, IDX_W)], semr[s]))
            return cps

        def compute(i):
            s = i % 2
            cr, xr, nr = crows.at[s], xrows.at[s], nrows.at[s]

            def g_body(g, _):
                cidx = lane + g * L
                nrow0 = (lane + g * L) * NNEG

                def d_body(d, carry):
                    accp = carry[0]
                    accs = carry[1]
                    dsp = (jnp.full((L,), d, jnp.int32) + lane) & (D - 1)
                    c = plsc.load_gather(cr, [cidx, dsp])
                    x = plsc.load_gather(xr, [cidx, dsp])
                    accp = accp + c * x
                    accs = tuple(
                        accs[n]
                        + plsc.load_gather(nr, [nrow0 + n, dsp]) * c
                        for n in range(NNEG))
                    return (accp, accs)

                zero = jnp.zeros((L,), jnp.float32)
                accp, accs = lax.fori_loop(
                    0, D, d_body, (zero, (zero,) * NNEG))
                plsc.store_scatter(posb.at[s], [cidx], _sigmoid(accp))
                for n in range(NNEG):
                    plsc.store_scatter(negb.at[s], [nrow0 + n],
                                       _sigmoid(-accs[n]))
                return ()

            lax.fori_loop(0, ngroups, g_body, ())

        def fire_out(i):
            s = i % 2
            base = chunk_base(i)
            nbase = pl.multiple_of(base * NNEG, CB * NNEG)
            return [
                pltpu.async_copy(posb.at[s], pos_hbm.at[pl.ds(base, CB)],
                                 semo[s]),
                pltpu.async_copy(negb.at[s],
                                 neg_hbm.at[pl.ds(nbase, nneg_rows)],
                                 semo[s]),
            ]

        def drain(cps):
            for cp in cps:
                cp.wait()

        d_idx, d_rows, d_out = {}, {}, {}
        d_idx[0] = fire_idx(0)
        d_idx[1] = fire_idx(1)
        drain(d_idx.pop(0))
        d_rows[0] = fire_rows(0)
        for i in range(nchunk):
            if i + 1 < nchunk:
                drain(d_idx.pop(i + 1))
                d_rows[i + 1] = fire_rows(i + 1)
            drain(d_rows.pop(i))
            if i + 2 < nchunk:
                d_idx[i + 2] = fire_idx(i + 2)
            if i - 2 in d_out:
                drain(d_out.pop(i - 2))
            compute(i)
            d_out[i] = fire_out(i)
        for k in sorted(d_out):
            drain(d_out[k])

    return sc_kernel


def kernel(center_table, context_table, center_words, context_words,
           negative_words):
    V, D = center_table.shape
    B = center_words.shape[0]
    NNEG = negative_words.shape[1]
    FULL = V // 128
    pad = ((0, 0), (0, P - D))
    tail_x = jnp.pad(context_table[FULL * 128:], pad).reshape(-1)
    tr = _build_transpose(V, D)
    lin_x = tr(context_table.T, tail_x)
    fn = _build(V, D, B, NNEG)
    pos, neg_flat = fn(
        center_table,
        lin_x.reshape(V, P),
        center_words.astype(jnp.int32),
        context_words.astype(jnp.int32),
        negative_words.astype(jnp.int32).reshape(-1),
    )
    return pos, neg_flat.reshape(B, NNEG)
